# baseline (speedup 1.0000x reference)
import functools
import math

import jax
import jax.numpy as jnp
from jax import lax
from jax.experimental import pallas as pl
from jax.experimental.pallas import tpu as pltpu
from jax.experimental.pallas import tpu_sc as plsc

D_MODEL = 64
SCALE = 1.0 / math.sqrt(D_MODEL)

C = 200


def kernel(x, lut):
    B1, B2 = x.shape
    V, D = lut.shape
    lut2 = lut.reshape(V // 2, 2 * D)
    flat_idx = x.reshape(B1 * B2).astype(jnp.int32)
    jdx = flat_idx >> 1
    poff = (flat_idx & 1) << 6
    return _call(jdx, poff, lut2, B1, B2, D)


@functools.partial(jax.jit, static_argnums=(3, 4, 5))
def _call(jdx, poff, lut2, B1, B2, D):
    info = plsc.get_sparse_core_info()
    NC, NS = info.num_cores, info.num_subcores
    NW = NC * NS
    n_chunks = (B1 * B2) // (NW * C)
    n_super = n_chunks // 2
    mesh = plsc.VectorSubcoreMesh(core_axis_name="c", subcore_axis_name="s")

    def body(jdx_hbm, poff_hbm, table_hbm, out_hbm,
             jdx_v, poff_v, rows2_v, rows1_v, gsem, wsem):
        wid = lax.axis_index("s") * NC + lax.axis_index("c")
        base = wid * n_chunks
        iota16 = lax.iota(jnp.int32, 16)

        def fetch(b, ci):
            off = (base + ci) * C
            pltpu.sync_copy(jdx_hbm.at[pl.ds(off, C)], jdx_v[b])
            pltpu.sync_copy(poff_hbm.at[pl.ds(off, C)],
                            poff_v[b].at[pl.ds(0, C)])
            pltpu.async_copy(table_hbm.at[jdx_v[b]], rows2_v[b], gsem[b])

        def compact(b):
            def grp(g, carry):
                r0 = g * 16
                rvec = r0 + iota16
                mask = rvec < C
                pvec = poff_v[b][pl.ds(r0, 16)]
                for c in range(D):
                    v = plsc.load_gather(rows2_v[b], [rvec, pvec + c],
                                         mask=mask)
                    plsc.store_scatter(
                        rows1_v[b],
                        [rvec, jnp.full((16,), c, jnp.int32)],
                        v * SCALE, mask=mask)
                return carry

            lax.fori_loop(0, (C + 15) // 16, grp, 0)

        for b in range(2):
            fetch(b, b)

        def super_body(s, carry):
            for b in range(2):
                ci = s * 2 + b
                pltpu.make_async_copy(table_hbm.at[jdx_v[b]], rows2_v[b],
                                      gsem[b]).wait()

                @pl.when(ci >= 2)
                def _():
                    pltpu.make_async_copy(
                        rows1_v[b], out_hbm.at[base + ci - 2], wsem[b]).wait()

                compact(b)
                pltpu.async_copy(rows1_v[b], out_hbm.at[base + ci], wsem[b])
                fetch(b, ci + 2)
            return carry

        lax.fori_loop(0, n_super - 1, super_body, 0)

        for b in range(2):
            ci = n_chunks - 2 + b
            pltpu.make_async_copy(table_hbm.at[jdx_v[b]], rows2_v[b],
                                  gsem[b]).wait()
            pltpu.make_async_copy(rows1_v[b], out_hbm.at[base + ci - 2],
                                  wsem[b]).wait()
            compact(b)
            pltpu.async_copy(rows1_v[b], out_hbm.at[base + ci], wsem[b])
        for b in range(2):
            ci = n_chunks - 2 + b
            pltpu.make_async_copy(rows1_v[b], out_hbm.at[base + ci],
                                  wsem[b]).wait()

    return pl.kernel(
        body,
        mesh=mesh,
        compiler_params=pltpu.CompilerParams(use_tc_tiling_on_sc=True,
                                             needs_layout_passes=False),
        out_type=jax.ShapeDtypeStruct((B1, B2, D), jnp.float32),
        scratch_types=[
            [pltpu.VMEM((C,), jnp.int32) for _ in range(2)],
            [pltpu.VMEM((16 * ((C + 15) // 16),), jnp.int32)
             for _ in range(2)],
            [pltpu.VMEM((C, 2 * D), jnp.float32) for _ in range(2)],
            [pltpu.VMEM((C, D), jnp.float32) for _ in range(2)],
            [pltpu.SemaphoreType.DMA for _ in range(2)],
            [pltpu.SemaphoreType.DMA for _ in range(2)],
        ],
    )(jdx, poff, lut2)


# speedup vs baseline: 2.6899x; 2.6899x over previous
import functools
import math

import jax
import jax.numpy as jnp
from jax import lax
from jax.experimental import pallas as pl
from jax.experimental.pallas import tpu as pltpu
from jax.experimental.pallas import tpu_sc as plsc

D_MODEL = 64
SCALE = 1.0 / math.sqrt(D_MODEL)

C = 200


def kernel(x, lut):
    B1, B2 = x.shape
    V, D = lut.shape
    lutp = jnp.pad(lut, ((0, 0), (0, 2 * D - lut.shape[1])))
    flat_idx = x.reshape(B1 * B2).astype(jnp.int32)
    return _call(flat_idx, lutp, B1, B2, D)


@functools.partial(jax.jit, static_argnums=(2, 3, 4))
def _call(flat_idx, lutp, B1, B2, D):
    info = plsc.get_sparse_core_info()
    NC, NS = info.num_cores, info.num_subcores
    NW = NC * NS
    n_chunks = (B1 * B2) // (NW * C)
    mesh = plsc.VectorSubcoreMesh(core_axis_name="c", subcore_axis_name="s")

    def body(idx_hbm, table_hbm, out_hbm, idx_v, rows2_v, rows1_v, gsem, wsem):
        wid = lax.axis_index("s") * NC + lax.axis_index("c")
        base = wid * n_chunks

        def fetch(b, ci):
            off = (base + ci) * C
            pltpu.sync_copy(idx_hbm.at[pl.ds(off, C)], idx_v[b])
            pltpu.async_copy(table_hbm.at[idx_v[b]], rows2_v[b], gsem[b])

        def compact(b):
            def grp(i, carry):
                r0 = i * 4
                for u in range(4):
                    for j in range(D // 16):
                        src = rows2_v[b][r0 + u, pl.ds(j * 16, 16)]
                        rows1_v[b][r0 + u, pl.ds(j * 16, 16)] = src * SCALE
                return carry

            lax.fori_loop(0, C // 4, grp, 0)

        for b in range(2):
            fetch(b, b)

        def super_body(s, carry):
            for b in range(2):
                ci = s * 2 + b
                pltpu.make_async_copy(table_hbm.at[idx_v[b]], rows2_v[b],
                                      gsem[b]).wait()

                @pl.when(ci >= 2)
                def _():
                    pltpu.make_async_copy(
                        rows1_v[b], out_hbm.at[base + ci - 2], wsem[b]).wait()

                compact(b)
                pltpu.async_copy(rows1_v[b], out_hbm.at[base + ci], wsem[b])
                fetch(b, ci + 2)
            return carry

        lax.fori_loop(0, n_chunks // 2 - 1, super_body, 0)

        for b in range(2):
            ci = n_chunks - 2 + b
            pltpu.make_async_copy(table_hbm.at[idx_v[b]], rows2_v[b],
                                  gsem[b]).wait()
            pltpu.make_async_copy(rows1_v[b], out_hbm.at[base + ci - 2],
                                  wsem[b]).wait()
            compact(b)
            pltpu.async_copy(rows1_v[b], out_hbm.at[base + ci], wsem[b])
        for b in range(2):
            ci = n_chunks - 2 + b
            pltpu.make_async_copy(rows1_v[b], out_hbm.at[base + ci],
                                  wsem[b]).wait()

    return pl.kernel(
        body,
        mesh=mesh,
        compiler_params=pltpu.CompilerParams(use_tc_tiling_on_sc=True,
                                             needs_layout_passes=False),
        out_type=jax.ShapeDtypeStruct((B1, B2, D), jnp.float32),
        scratch_types=[
            [pltpu.VMEM((C,), jnp.int32) for _ in range(2)],
            [pltpu.VMEM((C, 2 * D), jnp.float32) for _ in range(2)],
            [pltpu.VMEM((C, D), jnp.float32) for _ in range(2)],
            [pltpu.SemaphoreType.DMA for _ in range(2)],
            [pltpu.SemaphoreType.DMA for _ in range(2)],
        ],
    )(flat_idx, lutp)
